# baseline (speedup 1.0000x reference)
import functools

import jax
import jax.numpy as jnp
from jax import lax
from jax.experimental import pallas as pl
from jax.experimental.pallas import tpu as pltpu
from jax.experimental.pallas import tpu_sc as plsc

N_ATOMS = 1_000_000
DIM = 128
CHUNK = 128
CHUNKF = CHUNK * DIM
N_FULL = N_ATOMS // CHUNK
TAIL = N_ATOMS - N_FULL * CHUNK
TABLE_F = 119 * DIM
NC = 2
NS = 16
NW = NC * NS
BASE_CHUNKS = N_FULL // NW
EXTRA = N_FULL - BASE_CHUNKS * NW
NBUF = 2
ROUNDS = -(-(BASE_CHUNKS + 1) // NBUF)
SLAB = (BASE_CHUNKS + 1) * CHUNK

_mesh = plsc.VectorSubcoreMesh(core_axis_name="c", subcore_axis_name="s")


@functools.partial(
    pl.kernel,
    mesh=_mesh,
    compiler_params=pltpu.CompilerParams(needs_layout_passes=False),
    out_type=jax.ShapeDtypeStruct((N_ATOMS * DIM,), jnp.float32),
    scratch_types=[
        pltpu.VMEM((SLAB,), jnp.int32),
        pltpu.VMEM((TABLE_F,), jnp.float32),
        pltpu.VMEM((CHUNKF,), jnp.float32),
        pltpu.VMEM((CHUNKF,), jnp.float32),
        pltpu.SemaphoreType.DMA,
        pltpu.SemaphoreType.DMA,
    ],
)
def _embed(idx_hbm, table_hbm, out_hbm, idx_v, table_v, rows0, rows1, s0, s1):
    rows = (rows0, rows1)
    ssem = (s0, s1)
    wid = lax.axis_index("s") * NC + lax.axis_index("c")
    n_my = jnp.where(wid < EXTRA, BASE_CHUNKS + 1, BASE_CHUNKS)
    start_chunk = wid * BASE_CHUNKS + jnp.minimum(wid, EXTRA)
    atom0 = start_chunk * CHUNK

    pltpu.sync_copy(table_hbm, table_v)
    pltpu.sync_copy(
        idx_hbm.at[pl.ds(atom0, BASE_CHUNKS * CHUNK)],
        idx_v.at[pl.ds(0, BASE_CHUNKS * CHUNK)],
    )

    @pl.when(wid < EXTRA)
    def _extra_idx():
        pltpu.sync_copy(
            idx_hbm.at[pl.ds(atom0 + BASE_CHUNKS * CHUNK, CHUNK)],
            idx_v.at[pl.ds(BASE_CHUNKS * CHUNK, CHUNK)],
        )

    @pl.when(wid == NW - 1)
    def _tail_idx():
        pltpu.sync_copy(
            idx_hbm.at[pl.ds(N_FULL * CHUNK, TAIL)],
            idx_v.at[pl.ds(BASE_CHUNKS * CHUNK, TAIL)],
        )

    lane = jnp.arange(16, dtype=jnp.int32)
    lane_rows = lane * DIM

    def _compute_chunk(v, b, ngroups):
        rows_b = rows[b]

        def gbody(g, carry):
            idx_vec = idx_v[pl.ds(v * CHUNK + g * 16, 16)]
            tbase = idx_vec * DIM
            sbase = lane_rows + g * (16 * DIM)
            for d in range(DIM):
                val = plsc.load_gather(table_v, [tbase + d])
                plsc.store_scatter(rows_b, [sbase + d], val)
            return carry

        lax.fori_loop(0, ngroups, gbody, 0)

    def _store(v, b):
        pltpu.async_copy(
            rows[b],
            out_hbm.at[pl.ds((start_chunk + v) * CHUNKF, CHUNKF)],
            ssem[b],
        )

    def _store_wait(v, b):
        pltpu.make_async_copy(
            rows[b],
            out_hbm.at[pl.ds((start_chunk + v) * CHUNKF, CHUNKF)],
            ssem[b],
        ).wait()

    def round_body(r, carry):
        for b in range(NBUF):
            v = r * NBUF + b

            @pl.when(v < n_my)
            def _(b=b, v=v):
                @pl.when(v >= NBUF)
                def _wait_prev():
                    _store_wait(v - NBUF, b)

                _compute_chunk(v, b, CHUNK // 16)
                _store(v, b)

        return carry

    lax.fori_loop(0, ROUNDS, round_body, 0)

    for b in range(NBUF):
        v = (ROUNDS - 1) * NBUF + b

        @pl.when(v < n_my)
        def _(b=b, v=v):
            _store_wait(v, b)

    @pl.when(wid == NW - 1)
    def _tail():
        _compute_chunk(BASE_CHUNKS, 0, TAIL // 16)
        pltpu.sync_copy(
            rows[0].at[pl.ds(0, TAIL * DIM)],
            out_hbm.at[pl.ds(N_FULL * CHUNKF, TAIL * DIM)],
        )


def kernel(Z, embeddings):
    out = _embed(Z.astype(jnp.int32), embeddings.reshape(-1))
    return out.reshape(N_ATOMS, DIM)


# speedup vs baseline: 4.8917x; 4.8917x over previous
import functools

import jax
import jax.numpy as jnp
from jax import lax
from jax.experimental import pallas as pl
from jax.experimental.pallas import tpu as pltpu
from jax.experimental.pallas import tpu_sc as plsc

N_ATOMS = 1_000_000
DIM = 128
CHUNK = 128
CHUNKF = CHUNK * DIM
N_FULL = N_ATOMS // CHUNK
TAIL = N_ATOMS - N_FULL * CHUNK
TABLE_F = 119 * DIM
NC = 2
NS = 16
NW = NC * NS
BASE_CHUNKS = N_FULL // NW
EXTRA = N_FULL - BASE_CHUNKS * NW
NBUF = 2
ROUNDS = -(-(BASE_CHUNKS + 1) // NBUF)
SLAB = (BASE_CHUNKS + 1) * CHUNK

_mesh = plsc.VectorSubcoreMesh(core_axis_name="c", subcore_axis_name="s")


@functools.partial(
    pl.kernel,
    mesh=_mesh,
    compiler_params=pltpu.CompilerParams(needs_layout_passes=False),
    out_type=jax.ShapeDtypeStruct((N_ATOMS * DIM,), jnp.float32),
    scratch_types=[
        pltpu.VMEM((SLAB,), jnp.int32),
        pltpu.VMEM((TABLE_F,), jnp.float32),
        pltpu.VMEM((CHUNKF,), jnp.float32),
        pltpu.VMEM((CHUNKF,), jnp.float32),
        pltpu.SemaphoreType.DMA,
        pltpu.SemaphoreType.DMA,
    ],
)
def _embed(idx_hbm, table_hbm, out_hbm, idx_v, table_v, rows0, rows1, s0, s1):
    rows = (rows0, rows1)
    ssem = (s0, s1)
    wid = lax.axis_index("s") * NC + lax.axis_index("c")
    n_my = jnp.where(wid < EXTRA, BASE_CHUNKS + 1, BASE_CHUNKS)
    start_chunk = wid * BASE_CHUNKS + jnp.minimum(wid, EXTRA)
    atom0 = start_chunk * CHUNK

    pltpu.sync_copy(table_hbm, table_v)
    pltpu.sync_copy(
        idx_hbm.at[pl.ds(atom0, BASE_CHUNKS * CHUNK)],
        idx_v.at[pl.ds(0, BASE_CHUNKS * CHUNK)],
    )

    @pl.when(wid < EXTRA)
    def _extra_idx():
        pltpu.sync_copy(
            idx_hbm.at[pl.ds(atom0 + BASE_CHUNKS * CHUNK, CHUNK)],
            idx_v.at[pl.ds(BASE_CHUNKS * CHUNK, CHUNK)],
        )

    @pl.when(wid == NW - 1)
    def _tail_idx():
        pltpu.sync_copy(
            idx_hbm.at[pl.ds(N_FULL * CHUNK, TAIL)],
            idx_v.at[pl.ds(BASE_CHUNKS * CHUNK, TAIL)],
        )

    def _compute_chunk(v, b, ngroups):
        rows_b = rows[b]

        def gbody(g, carry):
            zvec = idx_v[pl.ds(v * CHUNK + g * 16, 16)]
            rowb = zvec * DIM
            for a in range(16):
                tb = rowb[a]
                ab = g * (16 * DIM) + a * DIM
                for c in range(0, DIM, 16):
                    rows_b[pl.ds(ab + c, 16)] = table_v[pl.ds(tb + c, 16)]
            return carry

        lax.fori_loop(0, ngroups, gbody, 0)

    def _store(v, b):
        pltpu.async_copy(
            rows[b],
            out_hbm.at[pl.ds((start_chunk + v) * CHUNKF, CHUNKF)],
            ssem[b],
        )

    def _store_wait(v, b):
        pltpu.make_async_copy(
            rows[b],
            out_hbm.at[pl.ds((start_chunk + v) * CHUNKF, CHUNKF)],
            ssem[b],
        ).wait()

    def round_body(r, carry):
        for b in range(NBUF):
            v = r * NBUF + b

            @pl.when(v < n_my)
            def _(b=b, v=v):
                @pl.when(v >= NBUF)
                def _wait_prev():
                    _store_wait(v - NBUF, b)

                _compute_chunk(v, b, CHUNK // 16)
                _store(v, b)

        return carry

    lax.fori_loop(0, ROUNDS, round_body, 0)

    for b in range(NBUF):
        v = (ROUNDS - 1) * NBUF + b

        @pl.when(v < n_my)
        def _(b=b, v=v):
            _store_wait(v, b)

    @pl.when(wid == NW - 1)
    def _tail():
        _compute_chunk(BASE_CHUNKS, 0, TAIL // 16)
        pltpu.sync_copy(
            rows[0].at[pl.ds(0, TAIL * DIM)],
            out_hbm.at[pl.ds(N_FULL * CHUNKF, TAIL * DIM)],
        )


def kernel(Z, embeddings):
    out = _embed(Z.astype(jnp.int32), embeddings.reshape(-1))
    return out.reshape(N_ATOMS, DIM)


# speedup vs baseline: 21.1509x; 4.3238x over previous
import functools

import jax
import jax.numpy as jnp
from jax import lax
from jax.experimental import pallas as pl
from jax.experimental.pallas import tpu as pltpu
from jax.experimental.pallas import tpu_sc as plsc

N_ATOMS = 1_000_000
DIM = 128
CHUNK = 128
N_FULL = N_ATOMS // CHUNK
TAIL = N_ATOMS - N_FULL * CHUNK
N_Z = 119
NC = 2
NS = 16
NW = NC * NS
BASE_CHUNKS = N_FULL // NW
EXTRA = N_FULL - BASE_CHUNKS * NW
NBUF = 5
ROUNDS = -(-(BASE_CHUNKS + 1) // NBUF)
SLAB = (BASE_CHUNKS + 1) * CHUNK

_mesh = plsc.VectorSubcoreMesh(core_axis_name="c", subcore_axis_name="s")


@functools.partial(
    pl.kernel,
    mesh=_mesh,
    out_type=jax.ShapeDtypeStruct((N_ATOMS, DIM), jnp.float32),
    scratch_types=[
        pltpu.VMEM((SLAB,), jnp.int32),
        pltpu.VMEM((NBUF, CHUNK, DIM), jnp.float32),
        pltpu.VMEM_SHARED((N_Z, DIM), jnp.float32),
        pltpu.SemaphoreType.DMA,
    ]
    + [pltpu.SemaphoreType.DMA] * NBUF
    + [pltpu.SemaphoreType.DMA] * NBUF,
)
def _embed(idx_hbm, table_hbm, out_hbm, idx_v, rows_v, table_sh, sem, *bsems):
    gsem = bsems[:NBUF]
    ssem = bsems[NBUF:]
    sid = lax.axis_index("s")
    wid = sid * NC + lax.axis_index("c")
    n_my = jnp.where(wid < EXTRA, BASE_CHUNKS + 1, BASE_CHUNKS)
    start_chunk = wid * BASE_CHUNKS + jnp.minimum(wid, EXTRA)
    atom0 = start_chunk * CHUNK

    @pl.when(sid == 0)
    def _stage_table():
        pltpu.sync_copy(table_hbm, table_sh)

    pltpu.sync_copy(
        idx_hbm.at[pl.ds(atom0, BASE_CHUNKS * CHUNK)],
        idx_v.at[pl.ds(0, BASE_CHUNKS * CHUNK)],
    )

    @pl.when(wid < EXTRA)
    def _extra_idx():
        pltpu.sync_copy(
            idx_hbm.at[pl.ds(atom0 + BASE_CHUNKS * CHUNK, CHUNK)],
            idx_v.at[pl.ds(BASE_CHUNKS * CHUNK, CHUNK)],
        )

    @pl.when(wid == NW - 1)
    def _tail_idx():
        pltpu.sync_copy(
            idx_hbm.at[pl.ds(N_FULL * CHUNK, TAIL)],
            idx_v.at[pl.ds(BASE_CHUNKS * CHUNK, TAIL)],
        )

    plsc.subcore_barrier()

    def _gather(v, b):
        pltpu.async_copy(
            table_sh.at[idx_v.at[pl.ds(v * CHUNK, CHUNK)]], rows_v.at[b], gsem[b]
        )

    def _gather_wait(v, b):
        pltpu.make_async_copy(
            table_sh.at[idx_v.at[pl.ds(v * CHUNK, CHUNK)]], rows_v.at[b], gsem[b]
        ).wait()

    def _store(v, b):
        pltpu.async_copy(
            rows_v.at[b], out_hbm.at[pl.ds((start_chunk + v) * CHUNK, CHUNK)], ssem[b]
        )

    def _store_wait(v, b):
        pltpu.make_async_copy(
            rows_v.at[b], out_hbm.at[pl.ds((start_chunk + v) * CHUNK, CHUNK)], ssem[b]
        ).wait()

    def round_body(r, carry):
        for b in range(NBUF):
            v = r * NBUF + b

            @pl.when(v < n_my)
            def _(b=b, v=v):
                @pl.when(r >= 1)
                def _wait_prev():
                    _store_wait(v - NBUF, b)

                _gather(v, b)

        for b in range(NBUF):
            v = r * NBUF + b

            @pl.when(v < n_my)
            def _(b=b, v=v):
                _gather_wait(v, b)
                _store(v, b)

        return carry

    lax.fori_loop(0, ROUNDS, round_body, 0)

    for b in range(NBUF):
        v = (ROUNDS - 1) * NBUF + b

        @pl.when(v < n_my)
        def _(b=b, v=v):
            _store_wait(v, b)

    @pl.when(wid == NW - 1)
    def _tail():
        base = N_FULL * CHUNK
        idx_t = idx_v.at[pl.ds(BASE_CHUNKS * CHUNK, TAIL)]
        rows_t = rows_v.at[0].at[pl.ds(0, TAIL)]
        pltpu.async_copy(table_sh.at[idx_t], rows_t, sem).wait()
        pltpu.sync_copy(rows_t, out_hbm.at[pl.ds(base, TAIL)])


def kernel(Z, embeddings):
    return _embed(Z.astype(jnp.int32), embeddings)


# speedup vs baseline: 21.3869x; 1.0112x over previous
import functools

import jax
import jax.numpy as jnp
from jax import lax
from jax.experimental import pallas as pl
from jax.experimental.pallas import tpu as pltpu
from jax.experimental.pallas import tpu_sc as plsc

N_ATOMS = 1_000_000
DIM = 128
CHUNK = 128
N_FULL = N_ATOMS // CHUNK
TAIL = N_ATOMS - N_FULL * CHUNK
N_Z = 119
NC = 2
NS = 16
NW = NC * NS
BASE_CHUNKS = N_FULL // NW
EXTRA = N_FULL - BASE_CHUNKS * NW
NBUF = 6
ROUNDS = -(-(BASE_CHUNKS + 1) // NBUF)
SLAB = (BASE_CHUNKS + 1) * CHUNK

_mesh = plsc.VectorSubcoreMesh(core_axis_name="c", subcore_axis_name="s")


@functools.partial(
    pl.kernel,
    mesh=_mesh,
    out_type=jax.ShapeDtypeStruct((N_ATOMS, DIM), jnp.float32),
    scratch_types=[
        pltpu.VMEM((SLAB,), jnp.int32),
        pltpu.VMEM((NBUF, CHUNK, DIM), jnp.float32),
        pltpu.VMEM_SHARED((N_Z, DIM), jnp.float32),
        pltpu.SemaphoreType.DMA,
    ]
    + [pltpu.SemaphoreType.DMA] * NBUF
    + [pltpu.SemaphoreType.DMA] * NBUF,
)
def _embed(idx_hbm, table_hbm, out_hbm, idx_v, rows_v, table_sh, sem, *bsems):
    gsem = bsems[:NBUF]
    ssem = bsems[NBUF:]
    sid = lax.axis_index("s")
    wid = sid * NC + lax.axis_index("c")
    n_my = jnp.where(wid < EXTRA, BASE_CHUNKS + 1, BASE_CHUNKS)
    start_chunk = wid * BASE_CHUNKS + jnp.minimum(wid, EXTRA)
    atom0 = start_chunk * CHUNK

    @pl.when(sid == 0)
    def _stage_table():
        pltpu.sync_copy(table_hbm, table_sh)

    pltpu.sync_copy(
        idx_hbm.at[pl.ds(atom0, BASE_CHUNKS * CHUNK)],
        idx_v.at[pl.ds(0, BASE_CHUNKS * CHUNK)],
    )

    @pl.when(wid < EXTRA)
    def _extra_idx():
        pltpu.sync_copy(
            idx_hbm.at[pl.ds(atom0 + BASE_CHUNKS * CHUNK, CHUNK)],
            idx_v.at[pl.ds(BASE_CHUNKS * CHUNK, CHUNK)],
        )

    @pl.when(wid == NW - 1)
    def _tail_idx():
        pltpu.sync_copy(
            idx_hbm.at[pl.ds(N_FULL * CHUNK, TAIL)],
            idx_v.at[pl.ds(BASE_CHUNKS * CHUNK, TAIL)],
        )

    plsc.subcore_barrier()

    def _gather(v, b):
        pltpu.async_copy(
            table_sh.at[idx_v.at[pl.ds(v * CHUNK, CHUNK)]], rows_v.at[b], gsem[b]
        )

    def _gather_wait(v, b):
        pltpu.make_async_copy(
            table_sh.at[idx_v.at[pl.ds(v * CHUNK, CHUNK)]], rows_v.at[b], gsem[b]
        ).wait()

    def _store(v, b):
        pltpu.async_copy(
            rows_v.at[b], out_hbm.at[pl.ds((start_chunk + v) * CHUNK, CHUNK)], ssem[b]
        )

    def _store_wait(v, b):
        pltpu.make_async_copy(
            rows_v.at[b], out_hbm.at[pl.ds((start_chunk + v) * CHUNK, CHUNK)], ssem[b]
        ).wait()

    def round_body(r, carry):
        for b in range(NBUF):
            v = r * NBUF + b

            @pl.when(v < n_my)
            def _(b=b, v=v):
                @pl.when(r >= 1)
                def _wait_prev():
                    _store_wait(v - NBUF, b)

                _gather(v, b)

        for b in range(NBUF):
            v = r * NBUF + b

            @pl.when(v < n_my)
            def _(b=b, v=v):
                _gather_wait(v, b)
                _store(v, b)

        return carry

    lax.fori_loop(0, ROUNDS, round_body, 0)

    for b in range(NBUF):
        v = (ROUNDS - 1) * NBUF + b
        vlast = jnp.where(v < n_my, v, v - NBUF)

        @pl.when((vlast >= 0) & (vlast < n_my))
        def _(b=b, vlast=vlast):
            _store_wait(vlast, b)

    @pl.when(wid == NW - 1)
    def _tail():
        base = N_FULL * CHUNK
        idx_t = idx_v.at[pl.ds(BASE_CHUNKS * CHUNK, TAIL)]
        rows_t = rows_v.at[0].at[pl.ds(0, TAIL)]
        pltpu.async_copy(table_sh.at[idx_t], rows_t, sem).wait()
        pltpu.sync_copy(rows_t, out_hbm.at[pl.ds(base, TAIL)])


def kernel(Z, embeddings):
    return _embed(Z.astype(jnp.int32), embeddings)
